# Initial kernel scaffold; baseline (speedup 1.0000x reference)
import jax
import jax.numpy as jnp
from jax.experimental import pallas as pl


def kernel(inputs, adj_mat, weight, loop_weight, bias):
    raise NotImplementedError("write your pallas kernel here")


# speedup vs baseline: 1.0742x; 1.0742x over previous
import jax
import jax.numpy as jnp
from jax.experimental import pallas as pl
from jax.experimental.pallas import tpu as pltpu


def _pick_bm(n: int, target: int = 400) -> int:
    best = 8
    for bm in range(8, target + 1, 8):
        if n % bm == 0:
            best = bm
    return best


def _gcn_kernel(bm, x_ref, w_ref, wl_ref, b_ref, adj_ref, out_ref, s_ref, l_ref):
    i = pl.program_id(0)

    @pl.when(i == 0)
    def _():
        x = x_ref[...]
        s_ref[...] = jnp.dot(x, w_ref[...], preferred_element_type=jnp.float32)
        l_ref[...] = (
            jnp.dot(x, wl_ref[...], preferred_element_type=jnp.float32)
            + b_ref[...]
        )

    out_ref[...] = (
        jnp.dot(adj_ref[...], s_ref[...], preferred_element_type=jnp.float32)
        + l_ref[pl.ds(i * bm, bm), :]
    )


def kernel(inputs, adj_mat, weight, loop_weight, bias):
    n, d_in = inputs.shape
    d_out = weight.shape[1]
    bm = _pick_bm(n)
    grid = (n // bm,)

    bias2d = bias.reshape(1, d_out)

    import functools

    return pl.pallas_call(
        functools.partial(_gcn_kernel, bm),
        grid=grid,
        in_specs=[
            pl.BlockSpec((n, d_in), lambda i: (0, 0)),
            pl.BlockSpec((d_in, d_out), lambda i: (0, 0)),
            pl.BlockSpec((d_in, d_out), lambda i: (0, 0)),
            pl.BlockSpec((1, d_out), lambda i: (0, 0)),
            pl.BlockSpec((bm, n), lambda i: (i, 0)),
        ],
        out_specs=pl.BlockSpec((bm, d_out), lambda i: (i, 0)),
        out_shape=jax.ShapeDtypeStruct((n, d_out), jnp.float32),
        scratch_shapes=[
            pltpu.VMEM((n, d_out), jnp.float32),
            pltpu.VMEM((n, d_out), jnp.float32),
        ],
    )(inputs, weight, loop_weight, bias2d, adj_mat)


# speedup vs baseline: 1.0791x; 1.0046x over previous
import jax
import jax.numpy as jnp
from jax.experimental import pallas as pl
from jax.experimental.pallas import tpu as pltpu


def _pick_bm(n: int, target: int = 200) -> int:
    best = 8
    for bm in range(8, target + 1, 8):
        if n % bm == 0:
            best = bm
    return best


def _gcn_kernel(bm, x_ref, w_ref, wl_ref, b_ref, adj_ref, out_ref, s_ref, l_ref):
    i = pl.program_id(0)

    @pl.when(i == 0)
    def _():
        x = x_ref[...]
        s_ref[...] = jnp.dot(x, w_ref[...], preferred_element_type=jnp.float32)
        l_ref[...] = (
            jnp.dot(x, wl_ref[...], preferred_element_type=jnp.float32)
            + b_ref[...]
        )

    out_ref[...] = (
        jnp.dot(adj_ref[...], s_ref[...], preferred_element_type=jnp.float32)
        + l_ref[pl.ds(i * bm, bm), :]
    )


def kernel(inputs, adj_mat, weight, loop_weight, bias):
    n, d_in = inputs.shape
    d_out = weight.shape[1]
    bm = _pick_bm(n)
    grid = (n // bm,)

    bias2d = bias.reshape(1, d_out)

    import functools

    return pl.pallas_call(
        functools.partial(_gcn_kernel, bm),
        grid=grid,
        in_specs=[
            pl.BlockSpec((n, d_in), lambda i: (0, 0)),
            pl.BlockSpec((d_in, d_out), lambda i: (0, 0)),
            pl.BlockSpec((d_in, d_out), lambda i: (0, 0)),
            pl.BlockSpec((1, d_out), lambda i: (0, 0)),
            pl.BlockSpec((bm, n), lambda i: (i, 0)),
        ],
        out_specs=pl.BlockSpec((bm, d_out), lambda i: (i, 0)),
        out_shape=jax.ShapeDtypeStruct((n, d_out), jnp.float32),
        scratch_shapes=[
            pltpu.VMEM((n, d_out), jnp.float32),
            pltpu.VMEM((n, d_out), jnp.float32),
        ],
    )(inputs, weight, loop_weight, bias2d, adj_mat)
